# Initial kernel scaffold; baseline (speedup 1.0000x reference)
import jax
import jax.numpy as jnp
from jax.experimental import pallas as pl


def kernel(x, edge_index, batch, W1, b1, W2, b2, Wfc, bfc):
    raise NotImplementedError("write your pallas kernel here")


# speedup vs baseline: 20.2359x; 20.2359x over previous
import jax
import jax.numpy as jnp
from jax import lax
from jax.experimental import pallas as pl
from jax.experimental.pallas import tpu as pltpu
from jax.experimental.pallas import tpu_sc as plsc

N = 10000
E = 320000
DIN = 128
H = 128
H2 = 64
C = 10
G = 64

NP = 10240
NC = 2
NS = 16
NW = NC * NS
EPT = E // NW
K = 80
CH = EPT // K
ROWS = NP // NS
DEGW = 8

BN = 512
NB = NP // BN


def _mesh():
    return plsc.VectorSubcoreMesh(core_axis_name="c", subcore_axis_name="s")


def _deg_body(dst_hbm, zeros_hbm, ones_hbm, out_hbm, didx, ones_v, acc):
    cid = lax.axis_index("c")
    sid = lax.axis_index("s")
    wid = cid * NS + sid
    pltpu.sync_copy(zeros_hbm, acc.at[pl.ds(sid * ROWS, ROWS)])
    pltpu.sync_copy(ones_hbm, ones_v)
    plsc.subcore_barrier()
    pltpu.sync_copy(dst_hbm.at[wid], didx)

    def chunk(i, _):
        pltpu.sync_copy(ones_v, acc.at[didx.at[i]], add=True)
        return 0
    lax.fori_loop(0, CH, chunk, 0)
    plsc.subcore_barrier()
    pltpu.sync_copy(acc.at[pl.ds(sid * ROWS, ROWS)],
                    out_hbm.at[cid, pl.ds(sid * ROWS, ROWS)])


def _agg_body_for(width):
    def body(src_hbm, dst_hbm, table_hbm, zeros_hbm, out_hbm, sidx, didx, rows,
             acc, sem):
        cid = lax.axis_index("c")
        sid = lax.axis_index("s")
        wid = cid * NS + sid
        pltpu.sync_copy(zeros_hbm, acc.at[pl.ds(sid * ROWS, ROWS)])
        plsc.subcore_barrier()
        pltpu.sync_copy(src_hbm.at[wid], sidx)
        pltpu.sync_copy(dst_hbm.at[wid], didx)

        def chunk(i, _):
            pltpu.async_copy(table_hbm.at[sidx.at[i]], rows, sem).wait()
            pltpu.sync_copy(rows, acc.at[didx.at[i]], add=True)
            return 0
        lax.fori_loop(0, CH, chunk, 0)
        plsc.subcore_barrier()
        pltpu.sync_copy(acc.at[pl.ds(sid * ROWS, ROWS)],
                        out_hbm.at[cid, pl.ds(sid * ROWS, ROWS)])
    return body


def _agg_call(width, src3, dst3, table):
    kern = pl.kernel(
        _agg_body_for(width),
        out_type=jax.ShapeDtypeStruct((NC, NP, width), jnp.float32),
        mesh=_mesh(),
        scratch_types=[
            pltpu.VMEM((CH, K), jnp.int32),
            pltpu.VMEM((CH, K), jnp.int32),
            pltpu.VMEM((K, width), jnp.float32),
            pltpu.VMEM_SHARED((NP, width), jnp.float32),
            pltpu.SemaphoreType.DMA,
        ],
        compiler_params=pltpu.CompilerParams(use_tc_tiling_on_sc=False),
    )
    return kern(src3, dst3, table, jnp.zeros((ROWS, width), jnp.float32))


def _dinv_block(deg_ref):
    d = deg_ref[0, :, 0:1] + deg_ref[1, :, 0:1] + 1.0
    return lax.rsqrt(d)


def _t1_body(x_ref, w_ref, deg_ref, o_ref):
    dinv = _dinv_block(deg_ref)
    h = jnp.dot(x_ref[...], w_ref[...], preferred_element_type=jnp.float32,
                precision=lax.Precision.HIGHEST)
    o_ref[...] = h * dinv


def _t2_body(agg_ref, hs1_ref, deg_ref, b_ref, w_ref, o_ref):
    dinv = _dinv_block(deg_ref)
    tot = agg_ref[0] + agg_ref[1] + hs1_ref[...]
    h1 = jnp.maximum(tot * dinv + b_ref[...], 0.0)
    h2 = jnp.dot(h1, w_ref[...], preferred_element_type=jnp.float32,
                 precision=lax.Precision.HIGHEST)
    o_ref[...] = h2 * dinv


def _t3_body(agg_ref, hs2_ref, deg_ref, b_ref, batch_ref, wfc_ref, bfc_ref,
             o_ref, pool_acc, cnt_acc):
    i = pl.program_id(0)
    dinv = _dinv_block(deg_ref)
    tot = agg_ref[0] + agg_ref[1] + hs2_ref[...]
    h2 = jnp.maximum(tot * dinv + b_ref[...], 0.0)
    gid = lax.broadcasted_iota(jnp.int32, (BN, G), 1)
    m = jnp.where(batch_ref[...] == gid, 1.0, 0.0)

    @pl.when(i == 0)
    def _init():
        pool_acc[...] = jnp.zeros_like(pool_acc)
        cnt_acc[...] = jnp.zeros_like(cnt_acc)

    dn = (((0,), (0,)), ((), ()))
    pool_acc[...] += lax.dot_general(m, h2, dn,
                                     preferred_element_type=jnp.float32,
                                     precision=lax.Precision.HIGHEST)
    cnt_acc[...] += lax.dot_general(m, jnp.ones((BN, 1), jnp.float32), dn,
                                    preferred_element_type=jnp.float32,
                                    precision=lax.Precision.HIGHEST)

    @pl.when(i == NB - 1)
    def _fin():
        pooled = pool_acc[...] / jnp.maximum(cnt_acc[...], 1.0)
        o_ref[...] = jnp.dot(pooled, wfc_ref[...],
                             preferred_element_type=jnp.float32,
                             precision=lax.Precision.HIGHEST) + bfc_ref[...]


def kernel(x, edge_index, batch, W1, b1, W2, b2, Wfc, bfc):
    x = x.astype(jnp.float32)
    ei = edge_index.astype(jnp.int32)
    src3 = ei[0].reshape(NW, CH, K)
    dst3 = ei[1].reshape(NW, CH, K)
    x_p = jnp.pad(x, ((0, NP - N), (0, 0)))
    batch_p = jnp.pad(batch.astype(jnp.int32), (0, NP - N),
                      constant_values=G).reshape(NP, 1)
    W1T = W1.T
    W2T = W2.T
    WfcT = Wfc.T
    b1r = b1.reshape(1, H)
    b2r = b2.reshape(1, H2)
    bfcr = bfc.reshape(1, C)

    degacc = pl.kernel(
        _deg_body,
        out_type=jax.ShapeDtypeStruct((NC, NP, DEGW), jnp.float32),
        mesh=_mesh(),
        scratch_types=[
            pltpu.VMEM((CH, K), jnp.int32),
            pltpu.VMEM((K, DEGW), jnp.float32),
            pltpu.VMEM_SHARED((NP, DEGW), jnp.float32),
        ],
        compiler_params=pltpu.CompilerParams(use_tc_tiling_on_sc=False),
    )(dst3, jnp.zeros((ROWS, DEGW), jnp.float32),
      jnp.ones((K, DEGW), jnp.float32))

    hs1 = pl.pallas_call(
        _t1_body,
        grid=(NB,),
        in_specs=[
            pl.BlockSpec((BN, DIN), lambda i: (i, 0)),
            pl.BlockSpec((DIN, H), lambda i: (0, 0)),
            pl.BlockSpec((NC, BN, DEGW), lambda i: (0, i, 0)),
        ],
        out_specs=pl.BlockSpec((BN, H), lambda i: (i, 0)),
        out_shape=jax.ShapeDtypeStruct((NP, H), jnp.float32),
    )(x_p, W1T, degacc)

    aggB = _agg_call(H, src3, dst3, hs1)

    hs2 = pl.pallas_call(
        _t2_body,
        grid=(NB,),
        in_specs=[
            pl.BlockSpec((NC, BN, H), lambda i: (0, i, 0)),
            pl.BlockSpec((BN, H), lambda i: (i, 0)),
            pl.BlockSpec((NC, BN, DEGW), lambda i: (0, i, 0)),
            pl.BlockSpec((1, H), lambda i: (0, 0)),
            pl.BlockSpec((H, H2), lambda i: (0, 0)),
        ],
        out_specs=pl.BlockSpec((BN, H2), lambda i: (i, 0)),
        out_shape=jax.ShapeDtypeStruct((NP, H2), jnp.float32),
    )(aggB, hs1, degacc, b1r, W2T)

    aggC = _agg_call(H2, src3, dst3, hs2)

    out = pl.pallas_call(
        _t3_body,
        grid=(NB,),
        in_specs=[
            pl.BlockSpec((NC, BN, H2), lambda i: (0, i, 0)),
            pl.BlockSpec((BN, H2), lambda i: (i, 0)),
            pl.BlockSpec((NC, BN, DEGW), lambda i: (0, i, 0)),
            pl.BlockSpec((1, H2), lambda i: (0, 0)),
            pl.BlockSpec((BN, 1), lambda i: (i, 0)),
            pl.BlockSpec((H2, C), lambda i: (0, 0)),
            pl.BlockSpec((1, C), lambda i: (0, 0)),
        ],
        out_specs=pl.BlockSpec((G, C), lambda i: (0, 0)),
        out_shape=jax.ShapeDtypeStruct((G, C), jnp.float32),
        scratch_shapes=[
            pltpu.VMEM((G, H2), jnp.float32),
            pltpu.VMEM((G, 1), jnp.float32),
        ],
    )(aggC, hs2, degacc, b2r, batch_p, WfcT, bfcr)
    return out


# speedup vs baseline: 29.1801x; 1.4420x over previous
import jax
import jax.numpy as jnp
from jax import lax
from jax.experimental import pallas as pl
from jax.experimental.pallas import tpu as pltpu
from jax.experimental.pallas import tpu_sc as plsc

N = 10000
E = 320000
DIN = 128
H = 128
H2 = 64
C = 10
G = 64

NP = 10240
NC = 2
NS = 16
NW = NC * NS
EPT = E // NW
K = 80
CH = EPT // K
ROWS = NP // NS
DEGW = 8

BN = 512
NB = NP // BN


def _mesh():
    return plsc.VectorSubcoreMesh(core_axis_name="c", subcore_axis_name="s")


def _deg_body(dst_hbm, zeros_hbm, ones_hbm, out_hbm, didx, ones_v, acc):
    cid = lax.axis_index("c")
    sid = lax.axis_index("s")
    wid = cid * NS + sid
    pltpu.sync_copy(zeros_hbm, acc.at[pl.ds(sid * ROWS, ROWS)])
    pltpu.sync_copy(ones_hbm, ones_v)
    plsc.subcore_barrier()
    pltpu.sync_copy(dst_hbm.at[wid], didx)

    def chunk(i, _):
        pltpu.sync_copy(ones_v, acc.at[didx.at[i]], add=True)
        return 0
    lax.fori_loop(0, CH, chunk, 0)
    plsc.subcore_barrier()
    pltpu.sync_copy(acc.at[pl.ds(sid * ROWS, ROWS)],
                    out_hbm.at[cid, pl.ds(sid * ROWS, ROWS)])


def _agg_body_for(width):
    def body(src_hbm, dst_hbm, table_hbm, zeros_hbm, out_hbm, sidx, didx,
             rows0, rows1, acc, sem):
        cid = lax.axis_index("c")
        sid = lax.axis_index("s")
        wid = cid * NS + sid
        pltpu.sync_copy(zeros_hbm, acc.at[pl.ds(sid * ROWS, ROWS)])
        plsc.subcore_barrier()
        pltpu.sync_copy(src_hbm.at[wid], sidx)
        pltpu.sync_copy(dst_hbm.at[wid], didx)

        pltpu.async_copy(table_hbm.at[sidx.at[0]], rows0, sem)
        pltpu.async_copy(table_hbm.at[sidx.at[1]], rows1, sem)

        def step(c, rows_b):
            pltpu.make_async_copy(table_hbm.at[sidx.at[c]], rows_b,
                                  sem).wait()
            pltpu.sync_copy(rows_b, acc.at[didx.at[c]], add=True)

            @pl.when(c + 2 < CH)
            def _():
                pltpu.async_copy(table_hbm.at[sidx.at[c + 2]], rows_b, sem)

        def pair(t, _):
            step(2 * t, rows0)
            step(2 * t + 1, rows1)
            return 0
        lax.fori_loop(0, (CH - 1) // 2, pair, 0)
        step(CH - 1, rows0)
        plsc.subcore_barrier()
        pltpu.sync_copy(acc.at[pl.ds(sid * ROWS, ROWS)],
                        out_hbm.at[cid, pl.ds(sid * ROWS, ROWS)])
    return body


def _agg_call(width, src3, dst3, table):
    kern = pl.kernel(
        _agg_body_for(width),
        out_type=jax.ShapeDtypeStruct((NC, NP, width), jnp.float32),
        mesh=_mesh(),
        scratch_types=[
            pltpu.VMEM((CH, K), jnp.int32),
            pltpu.VMEM((CH, K), jnp.int32),
            pltpu.VMEM((K, width), jnp.float32),
            pltpu.VMEM((K, width), jnp.float32),
            pltpu.VMEM_SHARED((NP, width), jnp.float32),
            pltpu.SemaphoreType.DMA,
        ],
        compiler_params=pltpu.CompilerParams(use_tc_tiling_on_sc=False),
    )
    return kern(src3, dst3, table, jnp.zeros((ROWS, width), jnp.float32))


def _dinv_block(deg_ref):
    d = deg_ref[0, :, 0:1] + deg_ref[1, :, 0:1] + 1.0
    return lax.rsqrt(d)


def _t1_body(x_ref, w_ref, deg_ref, o_ref):
    dinv = _dinv_block(deg_ref)
    h = jnp.dot(x_ref[...], w_ref[...], preferred_element_type=jnp.float32,
                precision=lax.Precision.HIGHEST)
    o_ref[...] = h * dinv


def _t2_body(agg_ref, hs1_ref, deg_ref, b_ref, w_ref, o_ref):
    dinv = _dinv_block(deg_ref)
    tot = agg_ref[0] + agg_ref[1] + hs1_ref[...]
    h1 = jnp.maximum(tot * dinv + b_ref[...], 0.0)
    h2 = jnp.dot(h1, w_ref[...], preferred_element_type=jnp.float32,
                 precision=lax.Precision.HIGHEST)
    o_ref[...] = h2 * dinv


def _t3_body(agg_ref, hs2_ref, deg_ref, b_ref, batch_ref, wfc_ref, bfc_ref,
             o_ref, pool_acc, cnt_acc):
    i = pl.program_id(0)
    dinv = _dinv_block(deg_ref)
    tot = agg_ref[0] + agg_ref[1] + hs2_ref[...]
    h2 = jnp.maximum(tot * dinv + b_ref[...], 0.0)
    gid = lax.broadcasted_iota(jnp.int32, (BN, G), 1)
    m = jnp.where(batch_ref[...] == gid, 1.0, 0.0)

    @pl.when(i == 0)
    def _init():
        pool_acc[...] = jnp.zeros_like(pool_acc)
        cnt_acc[...] = jnp.zeros_like(cnt_acc)

    dn = (((0,), (0,)), ((), ()))
    pool_acc[...] += lax.dot_general(m, h2, dn,
                                     preferred_element_type=jnp.float32,
                                     precision=lax.Precision.HIGHEST)
    cnt_acc[...] += lax.dot_general(m, jnp.ones((BN, 1), jnp.float32), dn,
                                    preferred_element_type=jnp.float32,
                                    precision=lax.Precision.HIGHEST)

    @pl.when(i == NB - 1)
    def _fin():
        pooled = pool_acc[...] / jnp.maximum(cnt_acc[...], 1.0)
        o_ref[...] = jnp.dot(pooled, wfc_ref[...],
                             preferred_element_type=jnp.float32,
                             precision=lax.Precision.HIGHEST) + bfc_ref[...]


def kernel(x, edge_index, batch, W1, b1, W2, b2, Wfc, bfc):
    x = x.astype(jnp.float32)
    ei = edge_index.astype(jnp.int32)
    src3 = ei[0].reshape(NW, CH, K)
    dst3 = ei[1].reshape(NW, CH, K)
    x_p = jnp.pad(x, ((0, NP - N), (0, 0)))
    batch_p = jnp.pad(batch.astype(jnp.int32), (0, NP - N),
                      constant_values=G).reshape(NP, 1)
    W1T = W1.T
    W2T = W2.T
    WfcT = Wfc.T
    b1r = b1.reshape(1, H)
    b2r = b2.reshape(1, H2)
    bfcr = bfc.reshape(1, C)

    degacc = pl.kernel(
        _deg_body,
        out_type=jax.ShapeDtypeStruct((NC, NP, DEGW), jnp.float32),
        mesh=_mesh(),
        scratch_types=[
            pltpu.VMEM((CH, K), jnp.int32),
            pltpu.VMEM((K, DEGW), jnp.float32),
            pltpu.VMEM_SHARED((NP, DEGW), jnp.float32),
        ],
        compiler_params=pltpu.CompilerParams(use_tc_tiling_on_sc=False),
    )(dst3, jnp.zeros((ROWS, DEGW), jnp.float32),
      jnp.ones((K, DEGW), jnp.float32))

    hs1 = pl.pallas_call(
        _t1_body,
        grid=(NB,),
        in_specs=[
            pl.BlockSpec((BN, DIN), lambda i: (i, 0)),
            pl.BlockSpec((DIN, H), lambda i: (0, 0)),
            pl.BlockSpec((NC, BN, DEGW), lambda i: (0, i, 0)),
        ],
        out_specs=pl.BlockSpec((BN, H), lambda i: (i, 0)),
        out_shape=jax.ShapeDtypeStruct((NP, H), jnp.float32),
    )(x_p, W1T, degacc)

    aggB = _agg_call(H, src3, dst3, hs1)

    hs2 = pl.pallas_call(
        _t2_body,
        grid=(NB,),
        in_specs=[
            pl.BlockSpec((NC, BN, H), lambda i: (0, i, 0)),
            pl.BlockSpec((BN, H), lambda i: (i, 0)),
            pl.BlockSpec((NC, BN, DEGW), lambda i: (0, i, 0)),
            pl.BlockSpec((1, H), lambda i: (0, 0)),
            pl.BlockSpec((H, H2), lambda i: (0, 0)),
        ],
        out_specs=pl.BlockSpec((BN, H2), lambda i: (i, 0)),
        out_shape=jax.ShapeDtypeStruct((NP, H2), jnp.float32),
    )(aggB, hs1, degacc, b1r, W2T)

    aggC = _agg_call(H2, src3, dst3, hs2)

    out = pl.pallas_call(
        _t3_body,
        grid=(NB,),
        in_specs=[
            pl.BlockSpec((NC, BN, H2), lambda i: (0, i, 0)),
            pl.BlockSpec((BN, H2), lambda i: (i, 0)),
            pl.BlockSpec((NC, BN, DEGW), lambda i: (0, i, 0)),
            pl.BlockSpec((1, H2), lambda i: (0, 0)),
            pl.BlockSpec((BN, 1), lambda i: (i, 0)),
            pl.BlockSpec((H2, C), lambda i: (0, 0)),
            pl.BlockSpec((1, C), lambda i: (0, 0)),
        ],
        out_specs=pl.BlockSpec((G, C), lambda i: (0, 0)),
        out_shape=jax.ShapeDtypeStruct((G, C), jnp.float32),
        scratch_shapes=[
            pltpu.VMEM((G, H2), jnp.float32),
            pltpu.VMEM((G, 1), jnp.float32),
        ],
    )(aggC, hs2, degacc, b2r, batch_p, WfcT, bfcr)
    return out
